# Initial kernel scaffold; baseline (speedup 1.0000x reference)
import jax
import jax.numpy as jnp
from jax.experimental import pallas as pl


def kernel(state, weight):
    raise NotImplementedError("write your pallas kernel here")


# speedup vs baseline: 1.2107x; 1.2107x over previous
import functools

import jax
import jax.numpy as jnp
from jax import lax
from jax.experimental import pallas as pl
from jax.experimental.pallas import tpu as pltpu
from jax.experimental.pallas import tpu_sc as plsc

B, N, D = 256, 4096, 32
L = 16
NW = 32
BPW = B // NW
VPB = N // L
NR = N // 128
DG = 8
NG = D // DG


def _resample_body(st_hbm, c_hbm, w_hbm, mask_hbm,
                   outs_hbm, outw_hbm,
                   c_v, idx_v, rw_v, mask_v, row_v, orow_v):
    wid = lax.axis_index("s") * 2 + lax.axis_index("c")
    iota = lax.iota(jnp.int32, L)

    pltpu.sync_copy(mask_hbm, mask_v)
    rw = jnp.full((L,), 1.0 / N, jnp.float32)

    def rwfill(j, carry):
        rw_v[j // 8, pl.ds((j % 8) * L, L)] = rw
        return carry
    lax.fori_loop(0, VPB, rwfill, 0, unroll=8)

    def count(t):
        i0 = ((t - 1.0) * 0.5).astype(jnp.int32)
        for _ in range(2):
            i0 -= ((2.0 * i0.astype(jnp.float32) + 1.0) > t).astype(jnp.int32)
        for _ in range(2):
            i0 += ((2.0 * (i0 + 1).astype(jnp.float32) + 1.0) <= t).astype(jnp.int32)
        return jnp.clip(i0 + 1, 0, N)

    def per_batch(l, _):
        b = wid * BPW + l
        mvec = plsc.load_gather(mask_v, [jnp.full((L,), b, jnp.int32)])
        masked_s = jnp.max(mvec)

        @pl.when(masked_s == 0)
        def _passthrough():
            pltpu.sync_copy(st_hbm.at[pl.ds(b * D, D)],
                            outs_hbm.at[pl.ds(b * D, D)])
            pltpu.sync_copy(w_hbm.at[b], outw_hbm.at[b])

        @pl.when(masked_s != 0)
        def _resample():
            pltpu.sync_copy(c_hbm.at[b], c_v)

            def zero_body(j, carry):
                idx_v[pl.ds(j * L, L)] = jnp.zeros((L,), jnp.int32)
                return carry
            lax.fori_loop(0, VPB, zero_body, 0, unroll=8)

            def scat_body(j, carry):
                cur = c_v[j // 8, pl.ds((j % 8) * L, L)] * 8192.0
                nm1 = jnp.full((L,), j * L - 1, jnp.int32) + iota
                valid = nm1 >= 0
                nm1c = jnp.maximum(nm1, 0)
                prevc = plsc.load_gather(
                    c_v, [nm1c >> 7, nm1c & 127])
                prev = jnp.where(valid, prevc * 8192.0, 0.0)
                ccur = count(cur)
                cprev = count(prev)
                ivec = jnp.full((L,), j * L, jnp.int32) + iota
                pos = jnp.minimum(cprev, N - 1)
                plsc.store_scatter(idx_v, [pos], ivec, mask=ccur > cprev)
                return carry
            lax.fori_loop(0, VPB, scat_body, 0, unroll=4)

            def cm_body(j, carry):
                v = idx_v[pl.ds(j * L, L)]
                s = jnp.maximum(plsc.cummax(v), jnp.full((L,), carry, jnp.int32))
                idx_v[pl.ds(j * L, L)] = s
                return jnp.max(s)
            lax.fori_loop(0, VPB, cm_body, jnp.int32(0))

            def g_group(g, carry):
                pltpu.sync_copy(st_hbm.at[pl.ds(b * D + g * DG, DG)], row_v)

                def perm_body(j, carry2):
                    idxv = idx_v[pl.ds(j * L, L)]
                    r_i = idxv >> 7
                    c_i = idxv & 127
                    row = j // 8
                    col = (j % 8) * L
                    for d in range(DG):
                        dvec = jnp.full((L,), d, jnp.int32)
                        vals = plsc.load_gather(row_v, [dvec, r_i, c_i])
                        orow_v[d, row, pl.ds(col, L)] = vals
                    return carry2
                lax.fori_loop(0, VPB, perm_body, 0)
                pltpu.sync_copy(orow_v, outs_hbm.at[pl.ds(b * D + g * DG, DG)])
                return carry
            lax.fori_loop(0, NG, g_group, 0)

            pltpu.sync_copy(rw_v, outw_hbm.at[b])

        return 0

    lax.fori_loop(0, BPW, per_batch, 0)


@functools.partial(
    pl.kernel,
    out_type=[
        jax.ShapeDtypeStruct((B * D, NR, 128), jnp.float32),
        jax.ShapeDtypeStruct((B, NR, 128), jnp.float32),
    ],
    mesh=plsc.VectorSubcoreMesh(core_axis_name="c", subcore_axis_name="s"),
    compiler_params=pltpu.CompilerParams(
        needs_layout_passes=False, use_tc_tiling_on_sc=False
    ),
    scratch_types=[
        pltpu.VMEM((NR, 128), jnp.float32),
        pltpu.VMEM((N,), jnp.int32),
        pltpu.VMEM((NR, 128), jnp.float32),
        pltpu.VMEM((B,), jnp.int32),
        pltpu.VMEM((DG, NR, 128), jnp.float32),
        pltpu.VMEM((DG, NR, 128), jnp.float32),
    ],
)
def _sc_resample(st_hbm, c_hbm, w_hbm, mask_hbm, outs_hbm, outw_hbm,
                 c_v, idx_v, rw_v, mask_v, row_v, orow_v):
    _resample_body(st_hbm, c_hbm, w_hbm, mask_hbm, outs_hbm, outw_hbm,
                   c_v, idx_v, rw_v, mask_v, row_v, orow_v)


def kernel(state, weight):
    ess = 1.0 / jnp.sum(weight * weight, axis=1)
    mask = (ess < (N / 2.0)).astype(jnp.int32)
    cdf = jnp.cumsum(weight, axis=1)
    c = cdf / cdf[:, -1:]
    st = jnp.swapaxes(state, 1, 2).reshape(B * D, NR, 128)
    c3 = c.reshape(B, NR, 128)
    w3 = weight.reshape(B, NR, 128)
    outs3, outw3 = _sc_resample(st, c3, w3, mask)
    out_state = jnp.swapaxes(outs3.reshape(B, D, N), 1, 2)
    out_weight = outw3.reshape(B, N)
    return out_state, out_weight
